# Initial kernel scaffold; baseline (speedup 1.0000x reference)
import jax
import jax.numpy as jnp
from jax.experimental import pallas as pl


def kernel(attn_bias, spatial_pos, x, spatial_table, virtual_dist):
    raise NotImplementedError("write your pallas kernel here")


# speedup vs baseline: 4.1227x; 4.1227x over previous
import functools

import jax
import jax.numpy as jnp
from jax import lax
from jax.experimental import pallas as pl
from jax.experimental.pallas import tpu as pltpu
from jax.experimental.pallas import tpu_sc as plsc

NUM_HEADS = 32
NUM_SPATIAL = 512
B, N = 64, 128
N1 = N + 1
IMG = N1 * N1
IMG_PAD = 16656
SP = N * N

_info = plsc.get_sparse_core_info()
NC, NS = _info.num_cores, _info.num_subcores
NW = NC * NS
B_PER_W = B // NW


def _sc_kernel(ab_hbm, sp_hbm, tab_hbm, vd_hbm, out_hbm,
               table_v, sp_v, ab2_v, img_v, t_v):
    wid = lax.axis_index("s") * NC + lax.axis_index("c")
    pltpu.sync_copy(tab_hbm, table_v)
    pltpu.sync_copy(vd_hbm, t_v)
    iota = lax.iota(jnp.int32, 16)
    col_idx0 = iota * N1

    for bb in range(B_PER_W):
        b = wid * B_PER_W + bb
        pltpu.sync_copy(sp_hbm.at[b], sp_v)
        pltpu.sync_copy(ab_hbm.at[b], ab2_v.at[pl.ds(0, IMG)])

        def dbl(c, carry):
            v = ab2_v[pl.ds(c * 16, 16)]
            ab2_v[pl.ds(c * 16, 16)] = v + v
            return carry
        lax.fori_loop(0, IMG_PAD // 16, dbl, 0)

        def h_body(h, carry):
            tval = t_v[pl.ds(h * 16, 16)]
            for c in range(9):
                av = ab2_v[pl.ds(c * 16, 16)]
                img_v[pl.ds(c * 16, 16)] = av + tval
            for c in range(8):
                idx = col_idx0 + (1 + c * 16) * N1
                av = plsc.load_gather(ab2_v, [idx])
                plsc.store_scatter(img_v, [idx], av + tval)
            hbase = h * NUM_SPATIAL

            def row_body(i, rc):
                sp_base = (i - 1) * N
                img_base = i * N1 + 1
                for c in range(8):
                    spv = sp_v[pl.ds(sp_base + c * 16, 16)]
                    tv = plsc.load_gather(table_v, [spv + hbase])
                    ii = iota + (img_base + c * 16)
                    av = plsc.load_gather(ab2_v, [ii])
                    plsc.store_scatter(img_v, [ii], av + tv)
                return rc
            lax.fori_loop(1, N1, row_body, 0)

            pltpu.sync_copy(img_v.at[pl.ds(0, IMG)], out_hbm.at[b * NUM_HEADS + h])
            return carry
        lax.fori_loop(0, NUM_HEADS, h_body, 0)


def kernel(attn_bias, spatial_pos, x, spatial_table, virtual_dist):
    del x
    ab = attn_bias.reshape(B, IMG)
    sp = spatial_pos.reshape(B, SP).astype(jnp.int32)
    tab = spatial_table.astype(jnp.float32).T.reshape(NUM_HEADS * NUM_SPATIAL)
    vd = jnp.repeat(virtual_dist.reshape(NUM_HEADS), 16)

    mesh = plsc.VectorSubcoreMesh(core_axis_name="c", subcore_axis_name="s")
    run = functools.partial(
        pl.kernel,
        mesh=mesh,
        out_type=jax.ShapeDtypeStruct((B * NUM_HEADS, IMG), jnp.float32),
        compiler_params=pltpu.CompilerParams(
            needs_layout_passes=False, use_tc_tiling_on_sc=False),
        scratch_types=[
            pltpu.VMEM((NUM_HEADS * NUM_SPATIAL,), jnp.float32),
            pltpu.VMEM((SP,), jnp.int32),
            pltpu.VMEM((IMG_PAD,), jnp.float32),
            pltpu.VMEM((IMG_PAD,), jnp.float32),
            pltpu.VMEM((NUM_HEADS * 16,), jnp.float32),
        ],
    )(_sc_kernel)
    out = run(ab, sp, tab, vd)
    return out.reshape(B, NUM_HEADS, N1, N1)


# speedup vs baseline: 5.5888x; 1.3556x over previous
import functools

import jax
import jax.numpy as jnp
from jax import lax
from jax.experimental import pallas as pl
from jax.experimental.pallas import tpu as pltpu
from jax.experimental.pallas import tpu_sc as plsc

NUM_HEADS = 32
NUM_SPATIAL = 512
B, N = 64, 128
N1 = N + 1
SPF = N * N1
ABF = N1 * N1
K = 4
RSEG = 32
NSEG = (NUM_HEADS // K) * (N // RSEG)

_info = plsc.get_sparse_core_info()
NC, NS = _info.num_cores, _info.num_subcores
NW = NC * NS
B_PER_W = B // NW


def _sc_kernel(ab_hbm, spsh_hbm, tab_hbm, vd_hbm, out_hbm,
               table_v, sp_v, ab2_v, t_v, r0_v, int_v,
               sem0, sem1, semr):
    wid = lax.axis_index("s") * NC + lax.axis_index("c")
    pltpu.sync_copy(tab_hbm, table_v)
    pltpu.sync_copy(vd_hbm, t_v)
    iota = lax.iota(jnp.int32, 16)
    m0 = jnp.where(iota == 0, 1.0, 0.0).astype(jnp.float32)

    def seg_body(s, carry):
        b = wid * B_PER_W + s // NSEG
        rem = lax.rem(s, NSEG)
        hblk = rem // (N // RSEG)
        rseg = lax.rem(rem, N // RSEG)
        parity = lax.rem(s, 2)

        @pl.when(jnp.logical_and(s >= NSEG, rem == 0))
        def _drain_r0():
            for _ in range(NUM_HEADS):
                pltpu.make_async_copy(
                    r0_v.at[0], out_hbm.at[b, 0, 0, :], semr).wait()

        @pl.when(rem == 0)
        def _setup():
            pltpu.sync_copy(spsh_hbm.at[b], sp_v.at[pl.ds(0, SPF)])
            pltpu.sync_copy(ab_hbm.at[b], ab2_v.at[pl.ds(0, ABF)])
            sp_v[pl.ds(SPF, 16)] = jnp.zeros((16,), jnp.int32)

            def dbl(c, c2):
                sl = pl.ds(c * 16, 16)
                v = ab2_v[sl]
                ab2_v[sl] = v + v
                return c2
            lax.fori_loop(0, 1041, dbl, 0)

            def edge_row(h, c2):
                tval = t_v[pl.ds(h * 16, 16)]
                for c in range(8):
                    sl = pl.ds(c * 16, 16)
                    r0_v[h, sl] = ab2_v[sl] + tval
                return c2
            lax.fori_loop(0, NUM_HEADS, edge_row, 0)
            a128 = plsc.load_gather(ab2_v, [jnp.full((16,), 128, jnp.int32)])
            for g in range(2):
                hv = iota + g * 16
                tl = plsc.load_gather(t_v, [hv * 16])
                plsc.store_scatter(
                    r0_v, [hv, jnp.full((16,), 128, jnp.int32)], a128 + tl)

            def fire_r0(h, c2):
                pltpu.make_async_copy(
                    r0_v.at[h], out_hbm.at[b, h, 0, :], semr).start()
                return c2
            lax.fori_loop(0, NUM_HEADS, fire_r0, 0)

        h0 = hblk * K
        rbase = rseg * RSEG
        dsts = [out_hbm.at[b, h0 + k, pl.ds(1 + rbase, RSEG), :]
                for k in range(K)]

        @pl.when(jnp.logical_and(s >= 2, parity == 0))
        def _wait0():
            for k in range(K):
                pltpu.make_async_copy(int_v.at[0, k], dsts[k], sem0).wait()

        @pl.when(jnp.logical_and(s >= 2, parity == 1))
        def _wait1():
            for k in range(K):
                pltpu.make_async_copy(int_v.at[1, k], dsts[k], sem1).wait()

        tmk = [t_v[pl.ds((h0 + k) * 16, 16)] * m0 for k in range(K)]

        def row_body(i, c2):
            oi = 1 + rbase + i
            spb = (oi - 1) * N1
            abb = oi * N1
            for c in range(8):
                sl16 = c * 16
                spv = sp_v[pl.ds(spb + sl16, 16)]
                a2 = ab2_v[pl.ds(abb + sl16, 16)]
                for k in range(K):
                    tv = plsc.load_gather(
                        table_v, [spv + (h0 + k) * NUM_SPATIAL])
                    val = a2 + tv
                    if c == 0:
                        val = val + tmk[k]
                    int_v[parity, k, i, pl.ds(sl16, 16)] = val
            return c2
        lax.fori_loop(0, RSEG, row_body, 0)

        c128 = jnp.full((16,), 128, jnp.int32)
        for g in range(2):
            rows = iota + g * 16
            spl = plsc.load_gather(sp_v, [(rbase + rows) * N1 + 128])
            a2l = plsc.load_gather(ab2_v, [(1 + rbase + rows) * N1 + 128])
            for k in range(K):
                tvl = plsc.load_gather(table_v, [spl + (h0 + k) * NUM_SPATIAL])
                plsc.store_scatter(
                    int_v,
                    [jnp.full((16,), parity, jnp.int32),
                     jnp.full((16,), k, jnp.int32), rows, c128],
                    a2l + tvl)

        @pl.when(parity == 0)
        def _fire0():
            for k in range(K):
                pltpu.make_async_copy(int_v.at[0, k], dsts[k], sem0).start()

        @pl.when(parity == 1)
        def _fire1():
            for k in range(K):
                pltpu.make_async_copy(int_v.at[1, k], dsts[k], sem1).start()

        return carry

    lax.fori_loop(0, B_PER_W * NSEG, seg_body, 0)

    b_last = wid * B_PER_W + (B_PER_W - 1)
    for p in range(2):
        sem = (sem0, sem1)[p]
        for k in range(K):
            dst = out_hbm.at[b_last, k, pl.ds(1, RSEG), :]
            pltpu.make_async_copy(int_v.at[p, k], dst, sem).wait()
    for _ in range(NUM_HEADS):
        pltpu.make_async_copy(r0_v.at[0], out_hbm.at[b_last, 0, 0, :],
                              semr).wait()


def kernel(attn_bias, spatial_pos, x, spatial_table, virtual_dist):
    del x
    spsh = jnp.pad(spatial_pos.astype(jnp.int32),
                   ((0, 0), (0, 0), (1, 0))).reshape(B, SPF)
    ab = attn_bias.reshape(B, ABF)
    tab = spatial_table.astype(jnp.float32).T.reshape(NUM_HEADS * NUM_SPATIAL)
    vd = jnp.repeat(virtual_dist.reshape(NUM_HEADS), 16)

    mesh = plsc.VectorSubcoreMesh(core_axis_name="c", subcore_axis_name="s")
    run = functools.partial(
        pl.kernel,
        mesh=mesh,
        out_type=jax.ShapeDtypeStruct((B, NUM_HEADS, N1, N1), jnp.float32),
        compiler_params=pltpu.CompilerParams(
            needs_layout_passes=False, use_tc_tiling_on_sc=False),
        scratch_types=[
            pltpu.VMEM((NUM_HEADS * NUM_SPATIAL,), jnp.float32),
            pltpu.VMEM((SPF + 16,), jnp.int32),
            pltpu.VMEM((ABF + 15,), jnp.float32),
            pltpu.VMEM((NUM_HEADS * 16,), jnp.float32),
            pltpu.VMEM((NUM_HEADS, N1), jnp.float32),
            pltpu.VMEM((2, K, RSEG, N1), jnp.float32),
            pltpu.SemaphoreType.DMA,
            pltpu.SemaphoreType.DMA,
            pltpu.SemaphoreType.DMA,
        ],
    )(_sc_kernel)
    return run(ab, spsh, tab, vd)
